# Initial kernel scaffold; baseline (speedup 1.0000x reference)
import jax
import jax.numpy as jnp
from jax.experimental import pallas as pl


def kernel(x, edge_index, W1_root, W1_nbr, b1, W2_root, W2_nbr, b2, Wa, ba, ctx, Wc1, bc1, Wc2, bc2):
    raise NotImplementedError("write your pallas kernel here")


# speedup vs baseline: 5.5357x; 5.5357x over previous
import functools

import jax
import jax.numpy as jnp
from jax import lax
from jax.experimental import pallas as pl
from jax.experimental.pallas import tpu as pltpu
from jax.experimental.pallas import tpu_sc as plsc

N = 100000
E = 1600000
D = 16
NC = 2
NS = 16
NW = NC * NS
CHUNK = 128

ROWS_PER_SUB = N // NS
ZROWS = 1250

EPW = E // NW
G_AB = EPW // CHUNK
T_AB = EPW - G_AB * CHUNK
EPS = E // NS
G_C = EPS // CHUNK
T_C = EPS - G_C * CHUNK

_MESH = plsc.VectorSubcoreMesh(core_axis_name="c", subcore_axis_name="s")
_SC_PARAMS = pltpu.CompilerParams(use_tc_tiling_on_sc=False)


def _zero_acc(acc, zeros_hbm, zrow_v, s):
    pltpu.sync_copy(zeros_hbm, zrow_v)
    base = s * ROWS_PER_SUB
    for z in range(ROWS_PER_SUB // ZROWS):
        pltpu.sync_copy(zrow_v, acc.at[pl.ds(base + z * ZROWS, ZROWS)])


def _writeout(acc, out_hbm, c, s):
    pltpu.sync_copy(acc.at[pl.ds(s * ROWS_PER_SUB, ROWS_PER_SUB)],
                    out_hbm.at[c, s])


def _sc_deg(edge_index, zeros_hbm, onehot_hbm):
    @functools.partial(
        pl.kernel,
        out_type=jax.ShapeDtypeStruct((NC, NS, ROWS_PER_SUB, D), jnp.float32),
        mesh=_MESH,
        compiler_params=_SC_PARAMS,
        scratch_types=[
            pltpu.VMEM_SHARED((N, D), jnp.float32),
            pltpu.VMEM((ZROWS, D), jnp.float32),
            pltpu.VMEM((CHUNK, D), jnp.float32),
            pltpu.VMEM((CHUNK,), jnp.int32),
            pltpu.VMEM((T_AB,), jnp.int32),
        ],
    )
    def k(ei_hbm, zeros_hbm_ref, onehot_hbm_ref, out_hbm,
          acc, zrow_v, ones_v, didx_v, didx_t):
        c = lax.axis_index("c")
        s = lax.axis_index("s")
        wid = s * NC + c
        base_w = wid * EPW
        pltpu.sync_copy(onehot_hbm_ref, ones_v)
        _zero_acc(acc, zeros_hbm_ref, zrow_v, s)
        plsc.subcore_barrier()

        @pl.loop(0, G_AB)
        def _(g):
            b = base_w + g * CHUNK
            pltpu.sync_copy(ei_hbm.at[pl.ds(E + b, CHUNK)], didx_v)
            pltpu.sync_copy(ones_v, acc.at[didx_v], add=True)

        pltpu.sync_copy(ei_hbm.at[pl.ds(E + base_w + G_AB * CHUNK, T_AB)],
                        didx_t)
        pltpu.sync_copy(ones_v.at[pl.ds(0, T_AB)], acc.at[didx_t], add=True)
        plsc.subcore_barrier()
        _writeout(acc, out_hbm, c, s)

    return k(edge_index, zeros_hbm, onehot_hbm)


def _sc_agg_l1(x, edge_index, zeros_hbm):
    @functools.partial(
        pl.kernel,
        out_type=jax.ShapeDtypeStruct((NC, NS, ROWS_PER_SUB, D), jnp.float32),
        mesh=_MESH,
        compiler_params=_SC_PARAMS,
        scratch_types=[
            pltpu.VMEM_SHARED((N, D), jnp.float32),
            pltpu.VMEM((ZROWS, D), jnp.float32),
            pltpu.VMEM((CHUNK, D), jnp.float32),
            pltpu.VMEM((T_AB, D), jnp.float32),
            pltpu.VMEM((CHUNK,), jnp.int32),
            pltpu.VMEM((CHUNK,), jnp.int32),
            pltpu.VMEM((T_AB,), jnp.int32),
            pltpu.VMEM((T_AB,), jnp.int32),
            pltpu.SemaphoreType.DMA,
        ],
    )
    def k(x_hbm, ei_hbm, zeros_hbm_ref, out_hbm,
          acc, zrow_v, rows_v, rows_t, sidx_v, didx_v, sidx_t, didx_t, sem):
        c = lax.axis_index("c")
        s = lax.axis_index("s")
        wid = s * NC + c
        base_w = wid * EPW
        _zero_acc(acc, zeros_hbm_ref, zrow_v, s)
        plsc.subcore_barrier()

        @pl.loop(0, G_AB)
        def _(g):
            b = base_w + g * CHUNK
            pltpu.sync_copy(ei_hbm.at[pl.ds(b, CHUNK)], sidx_v)
            pltpu.async_copy(x_hbm.at[sidx_v], rows_v, sem).wait()
            pltpu.sync_copy(ei_hbm.at[pl.ds(E + b, CHUNK)], didx_v)
            pltpu.sync_copy(rows_v, acc.at[didx_v], add=True)

        bt = base_w + G_AB * CHUNK
        pltpu.sync_copy(ei_hbm.at[pl.ds(bt, T_AB)], sidx_t)
        pltpu.async_copy(x_hbm.at[sidx_t], rows_t, sem).wait()
        pltpu.sync_copy(ei_hbm.at[pl.ds(E + bt, T_AB)], didx_t)
        pltpu.sync_copy(rows_t, acc.at[didx_t], add=True)
        plsc.subcore_barrier()
        _writeout(acc, out_hbm, c, s)

    return k(x, edge_index, zeros_hbm)


def _sc_agg_l2(hn, edge_index, zeros_hbm):
    @functools.partial(
        pl.kernel,
        out_type=jax.ShapeDtypeStruct((NC, NS, ROWS_PER_SUB, D), jnp.float32),
        mesh=_MESH,
        compiler_params=_SC_PARAMS,
        scratch_types=[
            pltpu.VMEM_SHARED((N, D), jnp.float32),
            pltpu.VMEM((ZROWS, D), jnp.float32),
            pltpu.VMEM((CHUNK, D), jnp.float32),
            pltpu.VMEM((T_C, D), jnp.float32),
            pltpu.VMEM((CHUNK,), jnp.int32),
            pltpu.VMEM((CHUNK,), jnp.int32),
            pltpu.VMEM((T_C,), jnp.int32),
            pltpu.VMEM((T_C,), jnp.int32),
            pltpu.SemaphoreType.DMA,
        ],
    )
    def k(hn_hbm, ei_hbm, zeros_hbm_ref, out_hbm,
          acc, zrow_v, rows_v, rows_t, sidx_v, didx_v, sidx_t, didx_t, sem):
        c = lax.axis_index("c")
        s = lax.axis_index("s")
        base_w = s * EPS
        _zero_acc(acc, zeros_hbm_ref, zrow_v, s)
        plsc.subcore_barrier()
        table = hn_hbm.at[c]

        @pl.loop(0, G_C)
        def _(g):
            b = base_w + g * CHUNK
            pltpu.sync_copy(ei_hbm.at[pl.ds(b, CHUNK)], sidx_v)
            pltpu.async_copy(table.at[sidx_v], rows_v, sem).wait()
            pltpu.sync_copy(ei_hbm.at[pl.ds(E + b, CHUNK)], didx_v)
            pltpu.sync_copy(rows_v, acc.at[didx_v], add=True)

        bt = base_w + G_C * CHUNK
        pltpu.sync_copy(ei_hbm.at[pl.ds(bt, T_C)], sidx_t)
        pltpu.async_copy(table.at[sidx_t], rows_t, sem).wait()
        pltpu.sync_copy(ei_hbm.at[pl.ds(E + bt, T_C)], didx_t)
        pltpu.sync_copy(rows_t, acc.at[didx_t], add=True)
        plsc.subcore_barrier()
        _writeout(acc, out_hbm, c, s)

    return k(hn, edge_index, zeros_hbm)


R1 = 2000
NB = N // R1


def _tc1_body(x_ref, a_ref, d_ref, w1r_ref, w1n_ref, b1_ref, w2r_ref,
              w2n_ref, hn_ref, hr_ref):
    deg = jnp.clip(d_ref[0, :, 0:1] + d_ref[1, :, 0:1], 1.0, None)
    mean1 = (a_ref[0] + a_ref[1]) / deg
    h = x_ref[...] @ w1r_ref[...] + mean1 @ w1n_ref[...] + b1_ref[...]
    h = jnp.maximum(h, 0.0)
    hn = h @ w2n_ref[...]
    hn_ref[0] = hn[:, :D]
    hn_ref[1] = hn[:, D:]
    hr_ref[...] = h @ w2r_ref[...]


def _tc1(x, agg1, deg, w1r_t, w1n_t, b1, w2r_t, w2n_t):
    full = lambda shape: pl.BlockSpec(shape, lambda i: tuple(0 for _ in shape))
    return pl.pallas_call(
        _tc1_body,
        grid=(NB,),
        in_specs=[
            pl.BlockSpec((R1, 16), lambda i: (i, 0)),
            pl.BlockSpec((NC, R1, D), lambda i: (0, i, 0)),
            pl.BlockSpec((NC, R1, D), lambda i: (0, i, 0)),
            full((16, 64)),
            full((16, 64)),
            full((1, 64)),
            full((64, 32)),
            full((64, 32)),
        ],
        out_specs=[
            pl.BlockSpec((NC, R1, D), lambda i: (0, i, 0)),
            pl.BlockSpec((R1, 32), lambda i: (i, 0)),
        ],
        out_shape=[
            jax.ShapeDtypeStruct((NC, N, D), jnp.float32),
            jax.ShapeDtypeStruct((N, 32), jnp.float32),
        ],
    )(x, agg1, deg, w1r_t, w1n_t, b1, w2r_t, w2n_t)


def _tc2_body(hr_ref, c_ref, d_ref, al_ref, b2_ref, wa_ref, ba_ref,
              ctx_ref, wc1_ref, bc1_ref, wc2_ref, bc2_ref, o_ref,
              m_ref, se_ref, acc_ref):
    i = pl.program_id(0)

    @pl.when(i == 0)
    def _():
        m_ref[0, 0] = -1e30
        se_ref[0, 0] = 0.0
        acc_ref[...] = jnp.zeros_like(acc_ref)

    deg = jnp.clip(d_ref[0, :, 0:1] + d_ref[1, :, 0:1], 1.0, None)
    mean2 = jnp.concatenate([c_ref[0], c_ref[1]], axis=1) / deg
    h2 = jnp.maximum(hr_ref[...] + mean2 + b2_ref[...], 0.0)
    scores = jnp.tanh(h2 @ wa_ref[...] + ba_ref[...])
    s = scores @ ctx_ref[...] + 0.4 * al_ref[...]

    m_old = m_ref[0, 0]
    m_new = jnp.maximum(m_old, jnp.max(s))
    scale = jnp.exp(m_old - m_new)
    w = jnp.exp(s - m_new)
    se_ref[0, 0] = se_ref[0, 0] * scale + jnp.sum(w)
    acc_ref[...] = acc_ref[...] * scale + jnp.sum(h2 * w, axis=0,
                                                  keepdims=True)
    m_ref[0, 0] = m_new

    @pl.when(i == NB - 1)
    def _():
        pooled = acc_ref[...] / (N * se_ref[0, 0])
        z = jnp.maximum(pooled @ wc1_ref[...] + bc1_ref[...], 0.0)
        o_ref[...] = jax.nn.sigmoid(z @ wc2_ref[...] + bc2_ref[...])


def _tc2(hr, agg2, deg, alerts, b2, wa_t, ba, ctx_col, wc1_t, bc1, wc2_t, bc2):
    full = lambda shape: pl.BlockSpec(shape, lambda i: tuple(0 for _ in shape))
    return pl.pallas_call(
        _tc2_body,
        grid=(NB,),
        in_specs=[
            pl.BlockSpec((R1, 32), lambda i: (i, 0)),
            pl.BlockSpec((NC, R1, D), lambda i: (0, i, 0)),
            pl.BlockSpec((NC, R1, D), lambda i: (0, i, 0)),
            pl.BlockSpec((R1, 1), lambda i: (i, 0)),
            full((1, 32)),
            full((32, 32)),
            full((1, 32)),
            full((32, 1)),
            full((32, 16)),
            full((1, 16)),
            full((16, 1)),
            full((1, 1)),
        ],
        out_specs=pl.BlockSpec((1, 1), lambda i: (0, 0)),
        out_shape=jax.ShapeDtypeStruct((1, 1), jnp.float32),
        scratch_shapes=[
            pltpu.SMEM((1, 1), jnp.float32),
            pltpu.SMEM((1, 1), jnp.float32),
            pltpu.VMEM((1, 32), jnp.float32),
        ],
    )(hr, agg2, deg, alerts, b2, wa_t, ba, ctx_col, wc1_t, bc1, wc2_t, bc2)


def kernel(x, edge_index, W1_root, W1_nbr, b1, W2_root, W2_nbr, b2,
           Wa, ba, ctx, Wc1, bc1, Wc2, bc2):
    zeros = jnp.zeros((ZROWS, D), jnp.float32)
    onehot = jnp.zeros((CHUNK, D), jnp.float32).at[:, 0].set(1.0)
    ei = edge_index.reshape(-1)

    deg = _sc_deg(ei, zeros, onehot).reshape(NC, N, D)
    agg1 = _sc_agg_l1(x, ei, zeros).reshape(NC, N, D)
    hn, hr = _tc1(x, agg1, deg, W1_root.T, W1_nbr.T, b1.reshape(1, -1),
                  W2_root.T, W2_nbr.T)
    agg2 = _sc_agg_l2(hn, ei, zeros).reshape(NC, N, D)
    out = _tc2(hr, agg2, deg, x[:, -1:], b2.reshape(1, -1), Wa.T,
               ba.reshape(1, -1), ctx.reshape(-1, 1), Wc1.T,
               bc1.reshape(1, -1), Wc2.T, bc2.reshape(1, -1))
    return out


# speedup vs baseline: 15.2966x; 2.7632x over previous
import functools

import jax
import jax.numpy as jnp
from jax import lax
from jax.experimental import pallas as pl
from jax.experimental.pallas import tpu as pltpu
from jax.experimental.pallas import tpu_sc as plsc

N = 100000
E = 1600000
D = 16
NC = 2
NS = 16
NW = NC * NS
CHUNK = 128
NCHUNK = E // CHUNK

ROWS_PER_SUB = N // NS

CPW = NCHUNK // NW
XTRA = NCHUNK - CPW * NW
GK_AB = 13
NG_AB = CPW // GK_AB

CPS = NCHUNK // NS
XTRA_C = NCHUNK - CPS * NS
GK_C = 11
NG_C = CPS // GK_C

_MESH = plsc.VectorSubcoreMesh(core_axis_name="c", subcore_axis_name="s")
_SC_PARAMS = pltpu.CompilerParams(use_tc_tiling_on_sc=False)


def _zero_acc(acc, zeros_hbm, s):
    pltpu.sync_copy(zeros_hbm, acc.at[pl.ds(s * ROWS_PER_SUB, ROWS_PER_SUB)])


def _writeout(acc, out_hbm, c, s):
    pltpu.sync_copy(acc.at[pl.ds(s * ROWS_PER_SUB, ROWS_PER_SUB)],
                    out_hbm.at[c, s])


_OUT_T = jax.ShapeDtypeStruct((NC, NS, ROWS_PER_SUB, D), jnp.float32)


def _sc_deg(ei3, zeros_hbm, onehot_hbm):
    @functools.partial(
        pl.kernel,
        out_type=_OUT_T,
        mesh=_MESH,
        compiler_params=_SC_PARAMS,
        scratch_types=[
            pltpu.VMEM_SHARED((N, D), jnp.float32),
            pltpu.VMEM((CHUNK, D), jnp.float32),
            pltpu.VMEM((GK_AB, CHUNK), jnp.int32),
            pltpu.SemaphoreType.DMA,
            pltpu.SemaphoreType.DMA,
        ],
    )
    def k(ei_hbm, zeros_hbm_ref, onehot_hbm_ref, out_hbm,
          acc, ones_v, didx_v, semi, sems):
        c = lax.axis_index("c")
        s = lax.axis_index("s")
        wid = s * NC + c
        cb = wid * CPW
        pltpu.sync_copy(onehot_hbm_ref, ones_v)
        _zero_acc(acc, zeros_hbm_ref, s)
        plsc.subcore_barrier()

        @pl.loop(0, NG_AB)
        def _(g):
            bc = cb + g * GK_AB
            pltpu.async_copy(ei_hbm.at[1, pl.ds(bc, GK_AB)], didx_v,
                             semi).wait()
            descs = [pltpu.async_copy(ones_v, acc.at[didx_v.at[j]], sems,
                                      add=True) for j in range(GK_AB)]
            for d in descs:
                d.wait()

        @pl.when(wid < XTRA)
        def _():
            pltpu.async_copy(ei_hbm.at[1, pl.ds(NW * CPW + wid, 1)],
                             didx_v.at[pl.ds(0, 1)], semi).wait()
            pltpu.async_copy(ones_v, acc.at[didx_v.at[0]], sems,
                             add=True).wait()

        plsc.subcore_barrier()
        _writeout(acc, out_hbm, c, s)

    return k(ei3, zeros_hbm, onehot_hbm)


def _sc_agg_l1(x, ei3, zeros_hbm):
    @functools.partial(
        pl.kernel,
        out_type=_OUT_T,
        mesh=_MESH,
        compiler_params=_SC_PARAMS,
        scratch_types=[
            pltpu.VMEM_SHARED((N, D), jnp.float32),
            pltpu.VMEM((GK_AB, CHUNK, D), jnp.float32),
            pltpu.VMEM((GK_AB, CHUNK), jnp.int32),
            pltpu.VMEM((GK_AB, CHUNK), jnp.int32),
            pltpu.SemaphoreType.DMA,
            pltpu.SemaphoreType.DMA,
            pltpu.SemaphoreType.DMA,
        ],
    )
    def k(x_hbm, ei_hbm, zeros_hbm_ref, out_hbm,
          acc, rows_v, sidx_v, didx_v, semi, semg, sems):
        c = lax.axis_index("c")
        s = lax.axis_index("s")
        wid = s * NC + c
        cb = wid * CPW
        _zero_acc(acc, zeros_hbm_ref, s)
        plsc.subcore_barrier()

        @pl.loop(0, NG_AB)
        def _(g):
            bc = cb + g * GK_AB
            d1 = pltpu.async_copy(ei_hbm.at[0, pl.ds(bc, GK_AB)], sidx_v,
                                  semi)
            d2 = pltpu.async_copy(ei_hbm.at[1, pl.ds(bc, GK_AB)], didx_v,
                                  semi)
            d1.wait()
            gs = [pltpu.async_copy(x_hbm.at[sidx_v.at[j]], rows_v.at[j],
                                   semg) for j in range(GK_AB)]
            d2.wait()
            for d in gs:
                d.wait()
            ss = [pltpu.async_copy(rows_v.at[j], acc.at[didx_v.at[j]], sems,
                                   add=True) for j in range(GK_AB)]
            for d in ss:
                d.wait()

        @pl.when(wid < XTRA)
        def _():
            bc = NW * CPW + wid
            d1 = pltpu.async_copy(ei_hbm.at[0, pl.ds(bc, 1)],
                                  sidx_v.at[pl.ds(0, 1)], semi)
            d2 = pltpu.async_copy(ei_hbm.at[1, pl.ds(bc, 1)],
                                  didx_v.at[pl.ds(0, 1)], semi)
            d1.wait()
            pltpu.async_copy(x_hbm.at[sidx_v.at[0]], rows_v.at[0],
                             semg).wait()
            d2.wait()
            pltpu.async_copy(rows_v.at[0], acc.at[didx_v.at[0]], sems,
                             add=True).wait()

        plsc.subcore_barrier()
        _writeout(acc, out_hbm, c, s)

    return k(x, ei3, zeros_hbm)


def _sc_agg_l2(hn, ei3, zeros_hbm):
    @functools.partial(
        pl.kernel,
        out_type=_OUT_T,
        mesh=_MESH,
        compiler_params=_SC_PARAMS,
        scratch_types=[
            pltpu.VMEM_SHARED((N, D), jnp.float32),
            pltpu.VMEM((GK_C, CHUNK, D), jnp.float32),
            pltpu.VMEM((GK_C, CHUNK), jnp.int32),
            pltpu.VMEM((GK_C, CHUNK), jnp.int32),
            pltpu.SemaphoreType.DMA,
            pltpu.SemaphoreType.DMA,
            pltpu.SemaphoreType.DMA,
        ],
    )
    def k(hn_hbm, ei_hbm, zeros_hbm_ref, out_hbm,
          acc, rows_v, sidx_v, didx_v, semi, semg, sems):
        c = lax.axis_index("c")
        s = lax.axis_index("s")
        cb = s * CPS
        _zero_acc(acc, zeros_hbm_ref, s)
        plsc.subcore_barrier()
        table = hn_hbm.at[c]

        @pl.loop(0, NG_C)
        def _(g):
            bc = cb + g * GK_C
            d1 = pltpu.async_copy(ei_hbm.at[0, pl.ds(bc, GK_C)], sidx_v,
                                  semi)
            d2 = pltpu.async_copy(ei_hbm.at[1, pl.ds(bc, GK_C)], didx_v,
                                  semi)
            d1.wait()
            gs = [pltpu.async_copy(table.at[sidx_v.at[j]], rows_v.at[j],
                                   semg) for j in range(GK_C)]
            d2.wait()
            for d in gs:
                d.wait()
            ss = [pltpu.async_copy(rows_v.at[j], acc.at[didx_v.at[j]], sems,
                                   add=True) for j in range(GK_C)]
            for d in ss:
                d.wait()

        @pl.when(s < XTRA_C)
        def _():
            bc = NS * CPS + s
            d1 = pltpu.async_copy(ei_hbm.at[0, pl.ds(bc, 1)],
                                  sidx_v.at[pl.ds(0, 1)], semi)
            d2 = pltpu.async_copy(ei_hbm.at[1, pl.ds(bc, 1)],
                                  didx_v.at[pl.ds(0, 1)], semi)
            d1.wait()
            pltpu.async_copy(table.at[sidx_v.at[0]], rows_v.at[0],
                             semg).wait()
            d2.wait()
            pltpu.async_copy(rows_v.at[0], acc.at[didx_v.at[0]], sems,
                             add=True).wait()

        plsc.subcore_barrier()
        _writeout(acc, out_hbm, c, s)

    return k(hn, ei3, zeros_hbm)


R1 = 2000
NB = N // R1


def _tc1_body(x_ref, a_ref, d_ref, w1r_ref, w1n_ref, b1_ref, w2r_ref,
              w2n_ref, hn_ref, hr_ref):
    deg = jnp.clip(d_ref[0, :, 0:1] + d_ref[1, :, 0:1], 1.0, None)
    mean1 = (a_ref[0] + a_ref[1]) / deg
    h = x_ref[...] @ w1r_ref[...] + mean1 @ w1n_ref[...] + b1_ref[...]
    h = jnp.maximum(h, 0.0)
    hn = h @ w2n_ref[...]
    hn_ref[0] = hn[:, :D]
    hn_ref[1] = hn[:, D:]
    hr_ref[...] = h @ w2r_ref[...]


def _tc1(x, agg1, deg, w1r_t, w1n_t, b1, w2r_t, w2n_t):
    full = lambda shape: pl.BlockSpec(shape, lambda i: tuple(0 for _ in shape))
    return pl.pallas_call(
        _tc1_body,
        grid=(NB,),
        in_specs=[
            pl.BlockSpec((R1, 16), lambda i: (i, 0)),
            pl.BlockSpec((NC, R1, D), lambda i: (0, i, 0)),
            pl.BlockSpec((NC, R1, D), lambda i: (0, i, 0)),
            full((16, 64)),
            full((16, 64)),
            full((1, 64)),
            full((64, 32)),
            full((64, 32)),
        ],
        out_specs=[
            pl.BlockSpec((NC, R1, D), lambda i: (0, i, 0)),
            pl.BlockSpec((R1, 32), lambda i: (i, 0)),
        ],
        out_shape=[
            jax.ShapeDtypeStruct((NC, N, D), jnp.float32),
            jax.ShapeDtypeStruct((N, 32), jnp.float32),
        ],
    )(x, agg1, deg, w1r_t, w1n_t, b1, w2r_t, w2n_t)


def _tc2_body(hr_ref, c_ref, d_ref, al_ref, b2_ref, wa_ref, ba_ref,
              ctx_ref, wc1_ref, bc1_ref, wc2_ref, bc2_ref, o_ref,
              m_ref, se_ref, acc_ref):
    i = pl.program_id(0)

    @pl.when(i == 0)
    def _():
        m_ref[0, 0] = -1e30
        se_ref[0, 0] = 0.0
        acc_ref[...] = jnp.zeros_like(acc_ref)

    deg = jnp.clip(d_ref[0, :, 0:1] + d_ref[1, :, 0:1], 1.0, None)
    mean2 = jnp.concatenate([c_ref[0], c_ref[1]], axis=1) / deg
    h2 = jnp.maximum(hr_ref[...] + mean2 + b2_ref[...], 0.0)
    scores = jnp.tanh(h2 @ wa_ref[...] + ba_ref[...])
    s = scores @ ctx_ref[...] + 0.4 * al_ref[...]

    m_old = m_ref[0, 0]
    m_new = jnp.maximum(m_old, jnp.max(s))
    scale = jnp.exp(m_old - m_new)
    w = jnp.exp(s - m_new)
    se_ref[0, 0] = se_ref[0, 0] * scale + jnp.sum(w)
    acc_ref[...] = acc_ref[...] * scale + jnp.sum(h2 * w, axis=0,
                                                  keepdims=True)
    m_ref[0, 0] = m_new

    @pl.when(i == NB - 1)
    def _():
        pooled = acc_ref[...] / (N * se_ref[0, 0])
        z = jnp.maximum(pooled @ wc1_ref[...] + bc1_ref[...], 0.0)
        o_ref[...] = jax.nn.sigmoid(z @ wc2_ref[...] + bc2_ref[...])


def _tc2(hr, agg2, deg, alerts, b2, wa_t, ba, ctx_col, wc1_t, bc1, wc2_t, bc2):
    full = lambda shape: pl.BlockSpec(shape, lambda i: tuple(0 for _ in shape))
    return pl.pallas_call(
        _tc2_body,
        grid=(NB,),
        in_specs=[
            pl.BlockSpec((R1, 32), lambda i: (i, 0)),
            pl.BlockSpec((NC, R1, D), lambda i: (0, i, 0)),
            pl.BlockSpec((NC, R1, D), lambda i: (0, i, 0)),
            pl.BlockSpec((R1, 1), lambda i: (i, 0)),
            full((1, 32)),
            full((32, 32)),
            full((1, 32)),
            full((32, 1)),
            full((32, 16)),
            full((1, 16)),
            full((16, 1)),
            full((1, 1)),
        ],
        out_specs=pl.BlockSpec((1, 1), lambda i: (0, 0)),
        out_shape=jax.ShapeDtypeStruct((1, 1), jnp.float32),
        scratch_shapes=[
            pltpu.SMEM((1, 1), jnp.float32),
            pltpu.SMEM((1, 1), jnp.float32),
            pltpu.VMEM((1, 32), jnp.float32),
        ],
    )(hr, agg2, deg, alerts, b2, wa_t, ba, ctx_col, wc1_t, bc1, wc2_t, bc2)


def kernel(x, edge_index, W1_root, W1_nbr, b1, W2_root, W2_nbr, b2,
           Wa, ba, ctx, Wc1, bc1, Wc2, bc2):
    zeros = jnp.zeros((ROWS_PER_SUB, D), jnp.float32)
    onehot = jnp.zeros((CHUNK, D), jnp.float32).at[:, 0].set(1.0)
    ei3 = edge_index.reshape(2, NCHUNK, CHUNK)

    deg = _sc_deg(ei3, zeros, onehot).reshape(NC, N, D)
    agg1 = _sc_agg_l1(x, ei3, zeros).reshape(NC, N, D)
    hn, hr = _tc1(x, agg1, deg, W1_root.T, W1_nbr.T, b1.reshape(1, -1),
                  W2_root.T, W2_nbr.T)
    agg2 = _sc_agg_l2(hn, ei3, zeros).reshape(NC, N, D)
    out = _tc2(hr, agg2, deg, x[:, -1:], b2.reshape(1, -1), Wa.T,
               ba.reshape(1, -1), ctx.reshape(-1, 1), Wc1.T,
               bc1.reshape(1, -1), Wc2.T, bc2.reshape(1, -1))
    return out


# speedup vs baseline: 15.7679x; 1.0308x over previous
import functools

import jax
import jax.numpy as jnp
from jax import lax
from jax.experimental import pallas as pl
from jax.experimental.pallas import tpu as pltpu
from jax.experimental.pallas import tpu_sc as plsc

N = 100000
E = 1600000
D = 16
NC = 2
NS = 16
NW = NC * NS
CHUNK = 128
NCHUNK = E // CHUNK

ROWS_PER_SUB = N // NS

CPW = NCHUNK // NW
XTRA = NCHUNK - CPW * NW
GK_AB = 13
NG_AB = CPW // GK_AB

CPS = NCHUNK // NS
XTRA_C = NCHUNK - CPS * NS
GK_C = 11
NG_C = CPS // GK_C

_MESH = plsc.VectorSubcoreMesh(core_axis_name="c", subcore_axis_name="s")
_SC_PARAMS = pltpu.CompilerParams(use_tc_tiling_on_sc=False)


def _zero_acc(acc, zeros_hbm, s):
    pltpu.sync_copy(zeros_hbm, acc.at[pl.ds(s * ROWS_PER_SUB, ROWS_PER_SUB)])


def _writeout(acc, out_hbm, c, s):
    pltpu.sync_copy(acc.at[pl.ds(s * ROWS_PER_SUB, ROWS_PER_SUB)],
                    out_hbm.at[c, s])


_OUT_T = jax.ShapeDtypeStruct((NC, NS, ROWS_PER_SUB, D), jnp.float32)


def _sc_deg(ei3, zeros_hbm, onehot_hbm):
    @functools.partial(
        pl.kernel,
        out_type=_OUT_T,
        mesh=_MESH,
        compiler_params=_SC_PARAMS,
        scratch_types=[
            pltpu.VMEM_SHARED((N, D), jnp.float32),
            pltpu.VMEM((CHUNK, D), jnp.float32),
            pltpu.VMEM((GK_AB, CHUNK), jnp.int32),
            pltpu.SemaphoreType.DMA,
            pltpu.SemaphoreType.DMA,
        ],
    )
    def k(ei_hbm, zeros_hbm_ref, onehot_hbm_ref, out_hbm,
          acc, ones_v, didx_v, semi, sems):
        c = lax.axis_index("c")
        s = lax.axis_index("s")
        wid = s * NC + c
        cb = wid * CPW
        pltpu.sync_copy(onehot_hbm_ref, ones_v)
        _zero_acc(acc, zeros_hbm_ref, s)
        plsc.subcore_barrier()

        @pl.loop(0, NG_AB)
        def _(g):
            bc = cb + g * GK_AB
            pltpu.async_copy(ei_hbm.at[1, pl.ds(bc, GK_AB)], didx_v,
                             semi).wait()
            descs = [pltpu.async_copy(ones_v, acc.at[didx_v.at[j]], sems,
                                      add=True) for j in range(GK_AB)]
            for d in descs:
                d.wait()

        @pl.when(wid < XTRA)
        def _():
            pltpu.async_copy(ei_hbm.at[1, pl.ds(NW * CPW + wid, 1)],
                             didx_v.at[pl.ds(0, 1)], semi).wait()
            pltpu.async_copy(ones_v, acc.at[didx_v.at[0]], sems,
                             add=True).wait()

        plsc.subcore_barrier()
        _writeout(acc, out_hbm, c, s)

    return k(ei3, zeros_hbm, onehot_hbm)


def _sc_agg_l1(x, ei3, zeros_hbm):
    @functools.partial(
        pl.kernel,
        out_type=_OUT_T,
        mesh=_MESH,
        compiler_params=_SC_PARAMS,
        scratch_types=[
            pltpu.VMEM_SHARED((N, D), jnp.float32),
            pltpu.VMEM((GK_AB, CHUNK, D), jnp.float32),
            pltpu.VMEM((GK_AB, CHUNK), jnp.int32),
            pltpu.VMEM((GK_AB, CHUNK), jnp.int32),
            pltpu.SemaphoreType.DMA,
            pltpu.SemaphoreType.DMA,
            pltpu.SemaphoreType.DMA,
        ],
    )
    def k(x_hbm, ei_hbm, zeros_hbm_ref, out_hbm,
          acc, rows_v, sidx_v, didx_v, semi, semg, sems):
        c = lax.axis_index("c")
        s = lax.axis_index("s")
        wid = s * NC + c
        cb = wid * CPW
        _zero_acc(acc, zeros_hbm_ref, s)
        plsc.subcore_barrier()

        @pl.loop(0, NG_AB)
        def _(g):
            bc = cb + g * GK_AB
            d1 = pltpu.async_copy(ei_hbm.at[0, pl.ds(bc, GK_AB)], sidx_v,
                                  semi)
            d2 = pltpu.async_copy(ei_hbm.at[1, pl.ds(bc, GK_AB)], didx_v,
                                  semi)
            d1.wait()
            gs = [pltpu.async_copy(x_hbm.at[sidx_v.at[j]], rows_v.at[j],
                                   semg) for j in range(GK_AB)]
            d2.wait()
            for d in gs:
                d.wait()
            ss = [pltpu.async_copy(rows_v.at[j], acc.at[didx_v.at[j]], sems,
                                   add=True) for j in range(GK_AB)]
            for d in ss:
                d.wait()

        @pl.when(wid < XTRA)
        def _():
            bc = NW * CPW + wid
            d1 = pltpu.async_copy(ei_hbm.at[0, pl.ds(bc, 1)],
                                  sidx_v.at[pl.ds(0, 1)], semi)
            d2 = pltpu.async_copy(ei_hbm.at[1, pl.ds(bc, 1)],
                                  didx_v.at[pl.ds(0, 1)], semi)
            d1.wait()
            pltpu.async_copy(x_hbm.at[sidx_v.at[0]], rows_v.at[0],
                             semg).wait()
            d2.wait()
            pltpu.async_copy(rows_v.at[0], acc.at[didx_v.at[0]], sems,
                             add=True).wait()

        plsc.subcore_barrier()
        _writeout(acc, out_hbm, c, s)

    return k(x, ei3, zeros_hbm)


def _sc_agg_l2(hn, ei3, zeros_hbm):
    @functools.partial(
        pl.kernel,
        out_type=_OUT_T,
        mesh=_MESH,
        compiler_params=_SC_PARAMS,
        scratch_types=[
            pltpu.VMEM_SHARED((N, D), jnp.float32),
            pltpu.VMEM((GK_C, CHUNK, D), jnp.float32),
            pltpu.VMEM((GK_C, CHUNK), jnp.int32),
            pltpu.VMEM((GK_C, CHUNK), jnp.int32),
            pltpu.SemaphoreType.DMA,
            pltpu.SemaphoreType.DMA,
            pltpu.SemaphoreType.DMA,
        ],
    )
    def k(hn_hbm, ei_hbm, zeros_hbm_ref, out_hbm,
          acc, rows_v, sidx_v, didx_v, semi, semg, sems):
        c = lax.axis_index("c")
        s = lax.axis_index("s")
        cb = s * CPS
        _zero_acc(acc, zeros_hbm_ref, s)
        plsc.subcore_barrier()
        table = hn_hbm.at[c]

        @pl.loop(0, NG_C)
        def _(g):
            bc = cb + g * GK_C
            d1 = pltpu.async_copy(ei_hbm.at[0, pl.ds(bc, GK_C)], sidx_v,
                                  semi)
            d2 = pltpu.async_copy(ei_hbm.at[1, pl.ds(bc, GK_C)], didx_v,
                                  semi)
            d1.wait()
            gs = [pltpu.async_copy(table.at[sidx_v.at[j]], rows_v.at[j],
                                   semg) for j in range(GK_C)]
            d2.wait()
            for d in gs:
                d.wait()
            ss = [pltpu.async_copy(rows_v.at[j], acc.at[didx_v.at[j]], sems,
                                   add=True) for j in range(GK_C)]
            for d in ss:
                d.wait()

        @pl.when(s < XTRA_C)
        def _():
            bc = NS * CPS + s
            d1 = pltpu.async_copy(ei_hbm.at[0, pl.ds(bc, 1)],
                                  sidx_v.at[pl.ds(0, 1)], semi)
            d2 = pltpu.async_copy(ei_hbm.at[1, pl.ds(bc, 1)],
                                  didx_v.at[pl.ds(0, 1)], semi)
            d1.wait()
            pltpu.async_copy(table.at[sidx_v.at[0]], rows_v.at[0],
                             semg).wait()
            d2.wait()
            pltpu.async_copy(rows_v.at[0], acc.at[didx_v.at[0]], sems,
                             add=True).wait()

        plsc.subcore_barrier()
        _writeout(acc, out_hbm, c, s)

    return k(hn, ei3, zeros_hbm)


LT = 4096
NBT = 25
N_PAD = LT * NBT


def _tc1_body(x_ref, a_ref, d_ref, w1r_ref, w1n_ref, b1_ref, w2r_ref,
              w2n_ref, hn_ref, hr_ref):
    deg = jnp.clip(d_ref[0] + d_ref[1], 1.0, None)
    mean1 = (a_ref[0] + a_ref[1]) / deg
    h = w1r_ref[...] @ x_ref[...] + w1n_ref[...] @ mean1 + b1_ref[...]
    h = jnp.maximum(h, 0.0)
    hn = w2n_ref[...] @ h
    hn_ref[0] = hn[:D]
    hn_ref[1] = hn[D:]
    hr_ref[...] = w2r_ref[...] @ h


def _tc1(xT, agg1T, degT, w1r, w1n, b1c, w2r, w2n):
    full = lambda shape: pl.BlockSpec(shape, lambda i: tuple(0 for _ in shape))
    return pl.pallas_call(
        _tc1_body,
        grid=(NBT,),
        in_specs=[
            pl.BlockSpec((16, LT), lambda i: (0, i)),
            pl.BlockSpec((NC, D, LT), lambda i: (0, 0, i)),
            pl.BlockSpec((NC, 1, LT), lambda i: (0, 0, i)),
            full((64, 16)),
            full((64, 16)),
            full((64, 1)),
            full((32, 64)),
            full((32, 64)),
        ],
        out_specs=[
            pl.BlockSpec((NC, D, LT), lambda i: (0, 0, i)),
            pl.BlockSpec((32, LT), lambda i: (0, i)),
        ],
        out_shape=[
            jax.ShapeDtypeStruct((NC, D, N_PAD), jnp.float32),
            jax.ShapeDtypeStruct((32, N_PAD), jnp.float32),
        ],
    )(xT, agg1T, degT, w1r, w1n, b1c, w2r, w2n)


def _tc2_body(hr_ref, c_ref, d_ref, al_ref, b2_ref, wa_ref, ba_ref,
              ctx_ref, wc1_ref, bc1_ref, wc2_ref, bc2_ref, o_ref,
              m_ref, se_ref, acc_ref):
    i = pl.program_id(0)

    @pl.when(i == 0)
    def _():
        m_ref[0, 0] = -1e30
        se_ref[0, 0] = 0.0
        acc_ref[...] = jnp.zeros_like(acc_ref)

    deg = jnp.clip(d_ref[0] + d_ref[1], 1.0, None)
    mean2 = jnp.concatenate([c_ref[0], c_ref[1]], axis=0) / deg
    h2 = jnp.maximum(hr_ref[...] + mean2 + b2_ref[...], 0.0)
    scores = jnp.tanh(wa_ref[...] @ h2 + ba_ref[...])
    s = ctx_ref[...] @ scores + 0.4 * al_ref[...]
    lane = jax.lax.broadcasted_iota(jnp.int32, (1, LT), 1)
    valid = (i * LT + lane) < N
    s = jnp.where(valid, s, -1e30)

    m_old = m_ref[0, 0]
    m_new = jnp.maximum(m_old, jnp.max(s))
    scale = jnp.exp(m_old - m_new)
    w = jnp.exp(s - m_new)
    se_ref[0, 0] = se_ref[0, 0] * scale + jnp.sum(w)
    acc_ref[...] = acc_ref[...] * scale + jnp.sum(h2 * w, axis=1,
                                                  keepdims=True)
    m_ref[0, 0] = m_new

    @pl.when(i == NBT - 1)
    def _():
        pooled = acc_ref[...] / (N * se_ref[0, 0])
        z = jnp.maximum(wc1_ref[...] @ pooled + bc1_ref[...], 0.0)
        o_ref[...] = jax.nn.sigmoid(wc2_ref[...] @ z + bc2_ref[...])


def _tc2(hrT, agg2T, degT, alT, b2c, wa, bac, ctx_row, wc1, bc1c, wc2, bc2c):
    full = lambda shape: pl.BlockSpec(shape, lambda i: tuple(0 for _ in shape))
    return pl.pallas_call(
        _tc2_body,
        grid=(NBT,),
        in_specs=[
            pl.BlockSpec((32, LT), lambda i: (0, i)),
            pl.BlockSpec((NC, D, LT), lambda i: (0, 0, i)),
            pl.BlockSpec((NC, 1, LT), lambda i: (0, 0, i)),
            pl.BlockSpec((1, LT), lambda i: (0, i)),
            full((32, 1)),
            full((32, 32)),
            full((32, 1)),
            full((1, 32)),
            full((16, 32)),
            full((16, 1)),
            full((1, 16)),
            full((1, 1)),
        ],
        out_specs=pl.BlockSpec((1, 1), lambda i: (0, 0)),
        out_shape=jax.ShapeDtypeStruct((1, 1), jnp.float32),
        scratch_shapes=[
            pltpu.SMEM((1, 1), jnp.float32),
            pltpu.SMEM((1, 1), jnp.float32),
            pltpu.VMEM((32, 1), jnp.float32),
        ],
    )(hrT, agg2T, degT, alT, b2c, wa, bac, ctx_row, wc1, bc1c, wc2, bc2c)


def kernel(x, edge_index, W1_root, W1_nbr, b1, W2_root, W2_nbr, b2,
           Wa, ba, ctx, Wc1, bc1, Wc2, bc2):
    zeros = jnp.zeros((ROWS_PER_SUB, D), jnp.float32)
    onehot = jnp.zeros((CHUNK, D), jnp.float32).at[:, 0].set(1.0)
    ei3 = edge_index.reshape(2, NCHUNK, CHUNK)

    deg = _sc_deg(ei3, zeros, onehot).reshape(NC, N, D)
    agg1 = _sc_agg_l1(x, ei3, zeros).reshape(NC, N, D)
    pad = N_PAD - N
    xT = jnp.pad(x.T, ((0, 0), (0, pad)))
    degT = jnp.pad(deg[:, :, 0], ((0, 0), (0, pad))).reshape(NC, 1, N_PAD)
    agg1T = jnp.pad(agg1.transpose(0, 2, 1), ((0, 0), (0, 0), (0, pad)))
    hnT, hrT = _tc1(xT, agg1T, degT, W1_root, W1_nbr, b1.reshape(-1, 1),
                    W2_root, W2_nbr)
    hn = hnT.transpose(0, 2, 1)
    agg2T = jnp.pad(
        _sc_agg_l2(hn, ei3, zeros).reshape(NC, N, D).transpose(0, 2, 1),
        ((0, 0), (0, 0), (0, pad)))
    out = _tc2(hrT, agg2T, degT, xT[D - 1:D], b2.reshape(-1, 1), Wa,
               ba.reshape(-1, 1), ctx.reshape(1, -1), Wc1,
               bc1.reshape(-1, 1), Wc2, bc2.reshape(-1, 1))
    return out


# speedup vs baseline: 24.4521x; 1.5508x over previous
import functools

import jax
import jax.numpy as jnp
from jax import lax
from jax.experimental import pallas as pl
from jax.experimental.pallas import tpu as pltpu
from jax.experimental.pallas import tpu_sc as plsc

N = 100000
E = 1600000
D = 16
NC = 2
NS = 16
NW = NC * NS
CHUNK = 128
NCHUNK = E // CHUNK

ROWS_PER_SUB = N // NS

CPW = NCHUNK // NW
XTRA = NCHUNK - CPW * NW
GK_AB = 13
NG_AB = CPW // GK_AB

CPS = NCHUNK // NS
XTRA_C = NCHUNK - CPS * NS
GK_C = 11
NG_C = CPS // GK_C

_MESH = plsc.VectorSubcoreMesh(core_axis_name="c", subcore_axis_name="s")
_SC_PARAMS = pltpu.CompilerParams(use_tc_tiling_on_sc=False)


def _zero_acc(acc, zeros_hbm, s):
    pltpu.sync_copy(zeros_hbm, acc.at[pl.ds(s * ROWS_PER_SUB, ROWS_PER_SUB)])


def _writeout(acc, out_hbm, c, s):
    pltpu.sync_copy(acc.at[pl.ds(s * ROWS_PER_SUB, ROWS_PER_SUB)],
                    out_hbm.at[c, s])


_OUT_T = jax.ShapeDtypeStruct((NC, NS, ROWS_PER_SUB, D), jnp.float32)


def _sc_deg(ei3, zeros_hbm, onehot_hbm):
    @functools.partial(
        pl.kernel,
        out_type=_OUT_T,
        mesh=_MESH,
        compiler_params=_SC_PARAMS,
        scratch_types=[
            pltpu.VMEM_SHARED((N, D), jnp.float32),
            pltpu.VMEM((CHUNK, D), jnp.float32),
            pltpu.VMEM((GK_AB, CHUNK), jnp.int32),
            pltpu.SemaphoreType.DMA,
            pltpu.SemaphoreType.DMA,
        ],
    )
    def k(ei_hbm, zeros_hbm_ref, onehot_hbm_ref, out_hbm,
          acc, ones_v, didx_v, semi, sems):
        c = lax.axis_index("c")
        s = lax.axis_index("s")
        wid = s * NC + c
        cb = wid * CPW
        pltpu.sync_copy(onehot_hbm_ref, ones_v)
        _zero_acc(acc, zeros_hbm_ref, s)
        plsc.subcore_barrier()

        @pl.loop(0, NG_AB)
        def _(g):
            bc = cb + g * GK_AB
            pltpu.async_copy(ei_hbm.at[1, pl.ds(bc, GK_AB)], didx_v,
                             semi).wait()
            descs = [pltpu.async_copy(ones_v, acc.at[didx_v.at[j]], sems,
                                      add=True) for j in range(GK_AB)]
            for d in descs:
                d.wait()

        @pl.when(wid < XTRA)
        def _():
            pltpu.async_copy(ei_hbm.at[1, pl.ds(NW * CPW + wid, 1)],
                             didx_v.at[pl.ds(0, 1)], semi).wait()
            pltpu.async_copy(ones_v, acc.at[didx_v.at[0]], sems,
                             add=True).wait()

        plsc.subcore_barrier()
        _writeout(acc, out_hbm, c, s)

    return k(ei3, zeros_hbm, onehot_hbm)


def _sc_agg_l1(x, ei3, zeros_hbm):
    @functools.partial(
        pl.kernel,
        out_type=_OUT_T,
        mesh=_MESH,
        compiler_params=_SC_PARAMS,
        scratch_types=[
            pltpu.VMEM_SHARED((N, D), jnp.float32),
            pltpu.VMEM((GK_AB, CHUNK, D), jnp.float32),
            pltpu.VMEM((GK_AB, CHUNK), jnp.int32),
            pltpu.VMEM((GK_AB, CHUNK), jnp.int32),
            pltpu.SemaphoreType.DMA,
            pltpu.SemaphoreType.DMA,
            pltpu.SemaphoreType.DMA,
        ],
    )
    def k(x_hbm, ei_hbm, zeros_hbm_ref, out_hbm,
          acc, rows_v, sidx_v, didx_v, semi, semg, sems):
        c = lax.axis_index("c")
        s = lax.axis_index("s")
        wid = s * NC + c
        cb = wid * CPW
        _zero_acc(acc, zeros_hbm_ref, s)
        plsc.subcore_barrier()

        @pl.loop(0, NG_AB)
        def _(g):
            bc = cb + g * GK_AB
            d1 = pltpu.async_copy(ei_hbm.at[0, pl.ds(bc, GK_AB)], sidx_v,
                                  semi)
            d2 = pltpu.async_copy(ei_hbm.at[1, pl.ds(bc, GK_AB)], didx_v,
                                  semi)
            d1.wait()
            gs = [pltpu.async_copy(x_hbm.at[sidx_v.at[j]], rows_v.at[j],
                                   semg) for j in range(GK_AB)]
            d2.wait()
            for d in gs:
                d.wait()
            ss = [pltpu.async_copy(rows_v.at[j], acc.at[didx_v.at[j]], sems,
                                   add=True) for j in range(GK_AB)]
            for d in ss:
                d.wait()

        @pl.when(wid < XTRA)
        def _():
            bc = NW * CPW + wid
            d1 = pltpu.async_copy(ei_hbm.at[0, pl.ds(bc, 1)],
                                  sidx_v.at[pl.ds(0, 1)], semi)
            d2 = pltpu.async_copy(ei_hbm.at[1, pl.ds(bc, 1)],
                                  didx_v.at[pl.ds(0, 1)], semi)
            d1.wait()
            pltpu.async_copy(x_hbm.at[sidx_v.at[0]], rows_v.at[0],
                             semg).wait()
            d2.wait()
            pltpu.async_copy(rows_v.at[0], acc.at[didx_v.at[0]], sems,
                             add=True).wait()

        plsc.subcore_barrier()
        _writeout(acc, out_hbm, c, s)

    return k(x, ei3, zeros_hbm)


def _sc_agg_l2(hn, ei3, zeros_hbm):
    @functools.partial(
        pl.kernel,
        out_type=_OUT_T,
        mesh=_MESH,
        compiler_params=_SC_PARAMS,
        scratch_types=[
            pltpu.VMEM_SHARED((N, D), jnp.float32),
            pltpu.VMEM((GK_C, CHUNK, D), jnp.float32),
            pltpu.VMEM((GK_C, CHUNK), jnp.int32),
            pltpu.VMEM((GK_C, CHUNK), jnp.int32),
            pltpu.SemaphoreType.DMA,
            pltpu.SemaphoreType.DMA,
            pltpu.SemaphoreType.DMA,
        ],
    )
    def k(hn_hbm, ei_hbm, zeros_hbm_ref, out_hbm,
          acc, rows_v, sidx_v, didx_v, semi, semg, sems):
        c = lax.axis_index("c")
        s = lax.axis_index("s")
        cb = s * CPS
        _zero_acc(acc, zeros_hbm_ref, s)
        plsc.subcore_barrier()
        table = hn_hbm.at[c]

        @pl.loop(0, NG_C)
        def _(g):
            bc = cb + g * GK_C
            d1 = pltpu.async_copy(ei_hbm.at[0, pl.ds(bc, GK_C)], sidx_v,
                                  semi)
            d2 = pltpu.async_copy(ei_hbm.at[1, pl.ds(bc, GK_C)], didx_v,
                                  semi)
            d1.wait()
            gs = [pltpu.async_copy(table.at[sidx_v.at[j]], rows_v.at[j],
                                   semg) for j in range(GK_C)]
            d2.wait()
            for d in gs:
                d.wait()
            ss = [pltpu.async_copy(rows_v.at[j], acc.at[didx_v.at[j]], sems,
                                   add=True) for j in range(GK_C)]
            for d in ss:
                d.wait()

        @pl.when(s < XTRA_C)
        def _():
            bc = NS * CPS + s
            d1 = pltpu.async_copy(ei_hbm.at[0, pl.ds(bc, 1)],
                                  sidx_v.at[pl.ds(0, 1)], semi)
            d2 = pltpu.async_copy(ei_hbm.at[1, pl.ds(bc, 1)],
                                  didx_v.at[pl.ds(0, 1)], semi)
            d1.wait()
            pltpu.async_copy(table.at[sidx_v.at[0]], rows_v.at[0],
                             semg).wait()
            d2.wait()
            pltpu.async_copy(rows_v.at[0], acc.at[didx_v.at[0]], sems,
                             add=True).wait()

        plsc.subcore_barrier()
        _writeout(acc, out_hbm, c, s)

    return k(hn, ei3, zeros_hbm)


RP = N * D // 128
RPAD = 12800
RB = 1600
NBP = RPAD // RB


def _tc1_body(x_ref, a_ref, d_ref, w1r_ref, w1n_ref, b1_ref, w2n0_ref,
              w2n1_ref, w2r_ref, hn_ref, hr_ref):
    deg = jnp.clip(d_ref[0] + d_ref[1], 1.0, None)
    mean1 = (a_ref[0] + a_ref[1]) / deg
    h = x_ref[...] @ w1r_ref[...] + mean1 @ w1n_ref[...] + b1_ref[...]
    h = jnp.maximum(h, 0.0)
    hn_ref[0] = h @ w2n0_ref[...]
    hn_ref[1] = h @ w2n1_ref[...]
    hr_ref[...] = h @ w2r_ref[...]


def _tc1(x_p, a_p, d_p, w1r_b, w1n_b, b1_t, w2n0_b, w2n1_b, w2r_b):
    full = lambda shape: pl.BlockSpec(shape, lambda i: tuple(0 for _ in shape))
    return pl.pallas_call(
        _tc1_body,
        grid=(NBP,),
        in_specs=[
            pl.BlockSpec((RB, 128), lambda i: (i, 0)),
            pl.BlockSpec((NC, RB, 128), lambda i: (0, i, 0)),
            pl.BlockSpec((NC, RB, 128), lambda i: (0, i, 0)),
            full((128, 512)),
            full((128, 512)),
            full((1, 512)),
            full((512, 128)),
            full((512, 128)),
            full((512, 256)),
        ],
        out_specs=[
            pl.BlockSpec((NC, RB, 128), lambda i: (0, i, 0)),
            pl.BlockSpec((RB, 256), lambda i: (i, 0)),
        ],
        out_shape=[
            jax.ShapeDtypeStruct((NC, RPAD, 128), jnp.float32),
            jax.ShapeDtypeStruct((RPAD, 256), jnp.float32),
        ],
    )(x_p, a_p, d_p, w1r_b, w1n_b, b1_t, w2n0_b, w2n1_b, w2r_b)


def _tc2_body(hr_ref, c_ref, d_ref, x_ref, s0_ref, s1_ref, b2_ref, wa_ref,
              ba_ref, ctxb_ref, alb_ref, fold_ref, wc1_ref, bc1_ref,
              wc2_ref, bc2_ref, o_ref, m_ref, se_ref, acc_ref):
    i = pl.program_id(0)

    @pl.when(i == 0)
    def _():
        m_ref[0, 0] = -1e30
        se_ref[0, 0] = 0.0
        acc_ref[...] = jnp.zeros_like(acc_ref)

    rdeg = 1.0 / jnp.clip(d_ref[0] + d_ref[1], 1.0, None)
    mean2 = (c_ref[0] * rdeg) @ s0_ref[...] + (c_ref[1] * rdeg) @ s1_ref[...]
    h2 = jnp.maximum(hr_ref[...] + mean2 + b2_ref[...], 0.0)
    t = jnp.tanh(h2 @ wa_ref[...] + ba_ref[...])
    s = t @ ctxb_ref[...] + x_ref[...] @ alb_ref[...]
    row = jax.lax.broadcasted_iota(jnp.int32, (RB, 256), 0) + i * RB
    s = jnp.where(row < RP, s, -1e30)

    m_old = m_ref[0, 0]
    m_new = jnp.maximum(m_old, jnp.max(s))
    scale = jnp.exp(m_old - m_new)
    w = jnp.exp(s - m_new)
    se_ref[0, 0] = se_ref[0, 0] * scale + jnp.sum(w) * (1.0 / 32.0)
    acc_ref[...] = acc_ref[...] * scale + (
        jnp.sum(h2 * w, axis=0, keepdims=True) @ fold_ref[...])
    m_ref[0, 0] = m_new

    @pl.when(i == NBP - 1)
    def _():
        pooled = acc_ref[...] / (N * se_ref[0, 0])
        z = jnp.maximum(pooled @ wc1_ref[...] + bc1_ref[...], 0.0)
        o_ref[...] = jax.nn.sigmoid(z @ wc2_ref[...] + bc2_ref[...])


def _tc2(hr_p, c_p, d_p, x_p, s0, s1, b2_t, wa_b, ba_t, ctx_b, al_b, fold,
         wc1_t, bc1_r, wc2_t, bc2_r):
    full = lambda shape: pl.BlockSpec(shape, lambda i: tuple(0 for _ in shape))
    return pl.pallas_call(
        _tc2_body,
        grid=(NBP,),
        in_specs=[
            pl.BlockSpec((RB, 256), lambda i: (i, 0)),
            pl.BlockSpec((NC, RB, 128), lambda i: (0, i, 0)),
            pl.BlockSpec((NC, RB, 128), lambda i: (0, i, 0)),
            pl.BlockSpec((RB, 128), lambda i: (i, 0)),
            full((128, 256)),
            full((128, 256)),
            full((1, 256)),
            full((256, 256)),
            full((1, 256)),
            full((256, 256)),
            full((128, 256)),
            full((256, 32)),
            full((32, 16)),
            full((1, 16)),
            full((16, 1)),
            full((1, 1)),
        ],
        out_specs=pl.BlockSpec((1, 1), lambda i: (0, 0)),
        out_shape=jax.ShapeDtypeStruct((1, 1), jnp.float32),
        scratch_shapes=[
            pltpu.SMEM((1, 1), jnp.float32),
            pltpu.SMEM((1, 1), jnp.float32),
            pltpu.VMEM((1, 32), jnp.float32),
        ],
    )(hr_p, c_p, d_p, x_p, s0, s1, b2_t, wa_b, ba_t, ctx_b, al_b, fold,
      wc1_t, bc1_r, wc2_t, bc2_r)


def _bd(m, k=8):
    return jax.scipy.linalg.block_diag(*([m] * k))


def kernel(x, edge_index, W1_root, W1_nbr, b1, W2_root, W2_nbr, b2,
           Wa, ba, ctx, Wc1, bc1, Wc2, bc2):
    zeros = jnp.zeros((ROWS_PER_SUB, D), jnp.float32)
    allones = jnp.ones((CHUNK, D), jnp.float32)
    ei3 = edge_index.reshape(2, NCHUNK, CHUNK)
    padr = ((0, RPAD - RP), (0, 0))

    deg = _sc_deg(ei3, zeros, allones)
    agg1 = _sc_agg_l1(x, ei3, zeros)
    x_p = jnp.pad(x.reshape(RP, 128), padr)
    d_p = jnp.pad(deg.reshape(NC, RP, 128), ((0, 0),) + padr)
    a_p = jnp.pad(agg1.reshape(NC, RP, 128), ((0, 0),) + padr)

    hn_p, hr_p = _tc1(
        x_p, a_p, d_p,
        _bd(W1_root.T), _bd(W1_nbr.T), jnp.tile(b1, 8).reshape(1, -1),
        _bd(W2_nbr.T[:, :D]), _bd(W2_nbr.T[:, D:]), _bd(W2_root.T))

    agg2 = _sc_agg_l2(hn_p.reshape(NC, RPAD * 8, D), ei3, zeros)
    c_p = jnp.pad(agg2.reshape(NC, RP, 128), ((0, 0),) + padr)

    eye16 = jnp.eye(D, dtype=jnp.float32)
    zz = jnp.zeros((D, D), jnp.float32)
    s0 = _bd(jnp.concatenate([eye16, zz], axis=1))
    s1 = _bd(jnp.concatenate([zz, eye16], axis=1))
    ctx_b = _bd(ctx.reshape(-1, 1) @ jnp.ones((1, 32), jnp.float32))
    al_b = _bd(jnp.zeros((D, 32), jnp.float32).at[D - 1, :].set(0.4))
    fold = jnp.tile(jnp.eye(32, dtype=jnp.float32), (8, 1))

    out = _tc2(hr_p, c_p, d_p, x_p, s0, s1,
               jnp.tile(b2, 8).reshape(1, -1), _bd(Wa.T),
               jnp.tile(ba, 8).reshape(1, -1), ctx_b, al_b, fold,
               Wc1.T, bc1.reshape(1, -1), Wc2.T, bc2.reshape(1, -1))
    return out


# speedup vs baseline: 27.3092x; 1.1168x over previous
import functools

import jax
import jax.numpy as jnp
from jax import lax
from jax.experimental import pallas as pl
from jax.experimental.pallas import tpu as pltpu
from jax.experimental.pallas import tpu_sc as plsc

N = 100000
E = 1600000
D = 16
NC = 2
NS = 16
NW = NC * NS
CHUNK = 128
NCHUNK = E // CHUNK

ROWS_PER_SUB = N // NS

CPW = NCHUNK // NW
XTRA = NCHUNK - CPW * NW
GK_AB = 13
NG_AB = CPW // GK_AB

CPS = NCHUNK // NS
XTRA_C = NCHUNK - CPS * NS
GK_C = 11
NG_C = CPS // GK_C

_MESH = plsc.VectorSubcoreMesh(core_axis_name="c", subcore_axis_name="s")
_SC_PARAMS = pltpu.CompilerParams(use_tc_tiling_on_sc=False)


def _zero_acc(acc, zeros_hbm, s):
    pltpu.sync_copy(zeros_hbm, acc.at[pl.ds(s * ROWS_PER_SUB, ROWS_PER_SUB)])


def _writeout(acc, out_hbm, c, s):
    pltpu.sync_copy(acc.at[pl.ds(s * ROWS_PER_SUB, ROWS_PER_SUB)],
                    out_hbm.at[c, s])


_OUT_T = jax.ShapeDtypeStruct((NC, NS, ROWS_PER_SUB, D), jnp.float32)


def _sc_deg(ei3, zeros_hbm, onehot_hbm):
    @functools.partial(
        pl.kernel,
        out_type=_OUT_T,
        mesh=_MESH,
        compiler_params=_SC_PARAMS,
        scratch_types=[
            pltpu.VMEM_SHARED((N, D), jnp.float32),
            pltpu.VMEM((CHUNK, D), jnp.float32),
            pltpu.VMEM((GK_AB, CHUNK), jnp.int32),
            pltpu.SemaphoreType.DMA,
            pltpu.SemaphoreType.DMA,
        ],
    )
    def k(ei_hbm, zeros_hbm_ref, onehot_hbm_ref, out_hbm,
          acc, ones_v, didx_v, semi, sems):
        c = lax.axis_index("c")
        s = lax.axis_index("s")
        wid = s * NC + c
        cb = wid * CPW
        pltpu.sync_copy(onehot_hbm_ref, ones_v)
        _zero_acc(acc, zeros_hbm_ref, s)
        plsc.subcore_barrier()

        @pl.loop(0, NG_AB)
        def _(g):
            bc = cb + g * GK_AB
            pltpu.async_copy(ei_hbm.at[1, pl.ds(bc, GK_AB)], didx_v,
                             semi).wait()
            descs = [pltpu.async_copy(ones_v, acc.at[didx_v.at[j]], sems,
                                      add=True) for j in range(GK_AB)]
            for d in descs:
                d.wait()

        @pl.when(wid < XTRA)
        def _():
            pltpu.async_copy(ei_hbm.at[1, pl.ds(NW * CPW + wid, 1)],
                             didx_v.at[pl.ds(0, 1)], semi).wait()
            pltpu.async_copy(ones_v, acc.at[didx_v.at[0]], sems,
                             add=True).wait()

        plsc.subcore_barrier()
        _writeout(acc, out_hbm, c, s)

    return k(ei3, zeros_hbm, onehot_hbm)


def _sc_agg_l1(x, ei3, zeros_hbm):
    @functools.partial(
        pl.kernel,
        out_type=_OUT_T,
        mesh=_MESH,
        compiler_params=_SC_PARAMS,
        scratch_types=[
            pltpu.VMEM_SHARED((N, D), jnp.float32),
            pltpu.VMEM((GK_AB, CHUNK, D), jnp.float32),
            pltpu.VMEM((GK_AB, CHUNK), jnp.int32),
            pltpu.VMEM((GK_AB, CHUNK), jnp.int32),
            pltpu.SemaphoreType.DMA,
            pltpu.SemaphoreType.DMA,
            pltpu.SemaphoreType.DMA,
        ],
    )
    def k(x_hbm, ei_hbm, zeros_hbm_ref, out_hbm,
          acc, rows_v, sidx_v, didx_v, semi, semg, sems):
        c = lax.axis_index("c")
        s = lax.axis_index("s")
        wid = s * NC + c
        cb = wid * CPW
        _zero_acc(acc, zeros_hbm_ref, s)
        plsc.subcore_barrier()

        @pl.loop(0, NG_AB)
        def _(g):
            bc = cb + g * GK_AB
            d1 = pltpu.async_copy(ei_hbm.at[0, pl.ds(bc, GK_AB)], sidx_v,
                                  semi)
            d2 = pltpu.async_copy(ei_hbm.at[1, pl.ds(bc, GK_AB)], didx_v,
                                  semi)
            d1.wait()
            gs = [pltpu.async_copy(x_hbm.at[sidx_v.at[j]], rows_v.at[j],
                                   semg) for j in range(GK_AB)]
            d2.wait()
            ss = []
            for j in range(GK_AB):
                gs[j].wait()
                ss.append(pltpu.async_copy(rows_v.at[j],
                                           acc.at[didx_v.at[j]], sems,
                                           add=True))
            for d in ss:
                d.wait()

        @pl.when(wid < XTRA)
        def _():
            bc = NW * CPW + wid
            d1 = pltpu.async_copy(ei_hbm.at[0, pl.ds(bc, 1)],
                                  sidx_v.at[pl.ds(0, 1)], semi)
            d2 = pltpu.async_copy(ei_hbm.at[1, pl.ds(bc, 1)],
                                  didx_v.at[pl.ds(0, 1)], semi)
            d1.wait()
            pltpu.async_copy(x_hbm.at[sidx_v.at[0]], rows_v.at[0],
                             semg).wait()
            d2.wait()
            pltpu.async_copy(rows_v.at[0], acc.at[didx_v.at[0]], sems,
                             add=True).wait()

        plsc.subcore_barrier()
        _writeout(acc, out_hbm, c, s)

    return k(x, ei3, zeros_hbm)


def _sc_agg_l2(hn, ei3, zeros_hbm):
    @functools.partial(
        pl.kernel,
        out_type=_OUT_T,
        mesh=_MESH,
        compiler_params=_SC_PARAMS,
        scratch_types=[
            pltpu.VMEM_SHARED((N, D), jnp.float32),
            pltpu.VMEM((GK_C, CHUNK, D), jnp.float32),
            pltpu.VMEM((GK_C, CHUNK), jnp.int32),
            pltpu.VMEM((GK_C, CHUNK), jnp.int32),
            pltpu.SemaphoreType.DMA,
            pltpu.SemaphoreType.DMA,
            pltpu.SemaphoreType.DMA,
        ],
    )
    def k(hn_hbm, ei_hbm, zeros_hbm_ref, out_hbm,
          acc, rows_v, sidx_v, didx_v, semi, semg, sems):
        c = lax.axis_index("c")
        s = lax.axis_index("s")
        cb = s * CPS
        _zero_acc(acc, zeros_hbm_ref, s)
        plsc.subcore_barrier()
        table = hn_hbm.at[c]

        @pl.loop(0, NG_C)
        def _(g):
            bc = cb + g * GK_C
            d1 = pltpu.async_copy(ei_hbm.at[0, pl.ds(bc, GK_C)], sidx_v,
                                  semi)
            d2 = pltpu.async_copy(ei_hbm.at[1, pl.ds(bc, GK_C)], didx_v,
                                  semi)
            d1.wait()
            gs = [pltpu.async_copy(table.at[sidx_v.at[j]], rows_v.at[j],
                                   semg) for j in range(GK_C)]
            d2.wait()
            ss = []
            for j in range(GK_C):
                gs[j].wait()
                ss.append(pltpu.async_copy(rows_v.at[j],
                                           acc.at[didx_v.at[j]], sems,
                                           add=True))
            for d in ss:
                d.wait()

        @pl.when(s < XTRA_C)
        def _():
            bc = NS * CPS + s
            d1 = pltpu.async_copy(ei_hbm.at[0, pl.ds(bc, 1)],
                                  sidx_v.at[pl.ds(0, 1)], semi)
            d2 = pltpu.async_copy(ei_hbm.at[1, pl.ds(bc, 1)],
                                  didx_v.at[pl.ds(0, 1)], semi)
            d1.wait()
            pltpu.async_copy(table.at[sidx_v.at[0]], rows_v.at[0],
                             semg).wait()
            d2.wait()
            pltpu.async_copy(rows_v.at[0], acc.at[didx_v.at[0]], sems,
                             add=True).wait()

        plsc.subcore_barrier()
        _writeout(acc, out_hbm, c, s)

    return k(hn, ei3, zeros_hbm)


RP = N * D // 128
RPAD = 12800
RB = 1600
NBP = RPAD // RB


def _tc1_body(x_ref, a_ref, d_ref, w1r_ref, w1n_ref, b1_ref, w2n0_ref,
              w2n1_ref, w2r_ref, hn_ref, hr_ref):
    deg = jnp.clip(d_ref[0] + d_ref[1], 1.0, None)
    mean1 = (a_ref[0] + a_ref[1]) / deg
    h = x_ref[...] @ w1r_ref[...] + mean1 @ w1n_ref[...] + b1_ref[...]
    h = jnp.maximum(h, 0.0)
    hn_ref[0] = h @ w2n0_ref[...]
    hn_ref[1] = h @ w2n1_ref[...]
    hr_ref[...] = h @ w2r_ref[...]


def _tc1(x_p, a_p, d_p, w1r_b, w1n_b, b1_t, w2n0_b, w2n1_b, w2r_b):
    full = lambda shape: pl.BlockSpec(shape, lambda i: tuple(0 for _ in shape))
    return pl.pallas_call(
        _tc1_body,
        grid=(NBP,),
        in_specs=[
            pl.BlockSpec((RB, 128), lambda i: (i, 0)),
            pl.BlockSpec((NC, RB, 128), lambda i: (0, i, 0)),
            pl.BlockSpec((NC, RB, 128), lambda i: (0, i, 0)),
            full((128, 512)),
            full((128, 512)),
            full((1, 512)),
            full((512, 128)),
            full((512, 128)),
            full((512, 256)),
        ],
        out_specs=[
            pl.BlockSpec((NC, RB, 128), lambda i: (0, i, 0)),
            pl.BlockSpec((RB, 256), lambda i: (i, 0)),
        ],
        out_shape=[
            jax.ShapeDtypeStruct((NC, RPAD, 128), jnp.float32),
            jax.ShapeDtypeStruct((RPAD, 256), jnp.float32),
        ],
    )(x_p, a_p, d_p, w1r_b, w1n_b, b1_t, w2n0_b, w2n1_b, w2r_b)


def _tc2_body(hr_ref, c_ref, d_ref, x_ref, s0_ref, s1_ref, b2_ref, wa_ref,
              ba_ref, ctxb_ref, alb_ref, fold_ref, wc1_ref, bc1_ref,
              wc2_ref, bc2_ref, o_ref, m_ref, se_ref, acc_ref):
    i = pl.program_id(0)

    @pl.when(i == 0)
    def _():
        m_ref[0, 0] = -1e30
        se_ref[0, 0] = 0.0
        acc_ref[...] = jnp.zeros_like(acc_ref)

    rdeg = 1.0 / jnp.clip(d_ref[0] + d_ref[1], 1.0, None)
    mean2 = (c_ref[0] * rdeg) @ s0_ref[...] + (c_ref[1] * rdeg) @ s1_ref[...]
    h2 = jnp.maximum(hr_ref[...] + mean2 + b2_ref[...], 0.0)
    t = jnp.tanh(h2 @ wa_ref[...] + ba_ref[...])
    s = t @ ctxb_ref[...] + x_ref[...] @ alb_ref[...]
    row = jax.lax.broadcasted_iota(jnp.int32, (RB, 256), 0) + i * RB
    s = jnp.where(row < RP, s, -1e30)

    m_old = m_ref[0, 0]
    m_new = jnp.maximum(m_old, jnp.max(s))
    scale = jnp.exp(m_old - m_new)
    w = jnp.exp(s - m_new)
    se_ref[0, 0] = se_ref[0, 0] * scale + jnp.sum(w) * (1.0 / 32.0)
    acc_ref[...] = acc_ref[...] * scale + (
        jnp.sum(h2 * w, axis=0, keepdims=True) @ fold_ref[...])
    m_ref[0, 0] = m_new

    @pl.when(i == NBP - 1)
    def _():
        pooled = acc_ref[...] / (N * se_ref[0, 0])
        z = jnp.maximum(pooled @ wc1_ref[...] + bc1_ref[...], 0.0)
        o_ref[...] = jax.nn.sigmoid(z @ wc2_ref[...] + bc2_ref[...])


def _tc2(hr_p, c_p, d_p, x_p, s0, s1, b2_t, wa_b, ba_t, ctx_b, al_b, fold,
         wc1_t, bc1_r, wc2_t, bc2_r):
    full = lambda shape: pl.BlockSpec(shape, lambda i: tuple(0 for _ in shape))
    return pl.pallas_call(
        _tc2_body,
        grid=(NBP,),
        in_specs=[
            pl.BlockSpec((RB, 256), lambda i: (i, 0)),
            pl.BlockSpec((NC, RB, 128), lambda i: (0, i, 0)),
            pl.BlockSpec((NC, RB, 128), lambda i: (0, i, 0)),
            pl.BlockSpec((RB, 128), lambda i: (i, 0)),
            full((128, 256)),
            full((128, 256)),
            full((1, 256)),
            full((256, 256)),
            full((1, 256)),
            full((256, 256)),
            full((128, 256)),
            full((256, 32)),
            full((32, 16)),
            full((1, 16)),
            full((16, 1)),
            full((1, 1)),
        ],
        out_specs=pl.BlockSpec((1, 1), lambda i: (0, 0)),
        out_shape=jax.ShapeDtypeStruct((1, 1), jnp.float32),
        scratch_shapes=[
            pltpu.SMEM((1, 1), jnp.float32),
            pltpu.SMEM((1, 1), jnp.float32),
            pltpu.VMEM((1, 32), jnp.float32),
        ],
    )(hr_p, c_p, d_p, x_p, s0, s1, b2_t, wa_b, ba_t, ctx_b, al_b, fold,
      wc1_t, bc1_r, wc2_t, bc2_r)


def _bd(m, k=8):
    return jax.scipy.linalg.block_diag(*([m] * k))


def kernel(x, edge_index, W1_root, W1_nbr, b1, W2_root, W2_nbr, b2,
           Wa, ba, ctx, Wc1, bc1, Wc2, bc2):
    zeros = jnp.zeros((ROWS_PER_SUB, D), jnp.float32)
    allones = jnp.ones((CHUNK, D), jnp.float32)
    ei3 = edge_index.reshape(2, NCHUNK, CHUNK)
    padr = ((0, RPAD - RP), (0, 0))

    deg = _sc_deg(ei3, zeros, allones)
    agg1 = _sc_agg_l1(x, ei3, zeros)
    x_p = jnp.pad(x.reshape(RP, 128), padr)
    d_p = jnp.pad(deg.reshape(NC, RP, 128), ((0, 0),) + padr)
    a_p = jnp.pad(agg1.reshape(NC, RP, 128), ((0, 0),) + padr)

    hn_p, hr_p = _tc1(
        x_p, a_p, d_p,
        _bd(W1_root.T), _bd(W1_nbr.T), jnp.tile(b1, 8).reshape(1, -1),
        _bd(W2_nbr.T[:, :D]), _bd(W2_nbr.T[:, D:]), _bd(W2_root.T))

    agg2 = _sc_agg_l2(hn_p.reshape(NC, RPAD * 8, D), ei3, zeros)
    c_p = jnp.pad(agg2.reshape(NC, RP, 128), ((0, 0),) + padr)

    eye16 = jnp.eye(D, dtype=jnp.float32)
    zz = jnp.zeros((D, D), jnp.float32)
    s0 = _bd(jnp.concatenate([eye16, zz], axis=1))
    s1 = _bd(jnp.concatenate([zz, eye16], axis=1))
    ctx_b = _bd(ctx.reshape(-1, 1) @ jnp.ones((1, 32), jnp.float32))
    al_b = _bd(jnp.zeros((D, 32), jnp.float32).at[D - 1, :].set(0.4))
    fold = jnp.tile(jnp.eye(32, dtype=jnp.float32), (8, 1))

    out = _tc2(hr_p, c_p, d_p, x_p, s0, s1,
               jnp.tile(b2, 8).reshape(1, -1), _bd(Wa.T),
               jnp.tile(ba, 8).reshape(1, -1), ctx_b, al_b, fold,
               Wc1.T, bc1.reshape(1, -1), Wc2.T, bc2.reshape(1, -1))
    return out
